# baseline (speedup 1.0000x reference)
import functools

import jax
import jax.numpy as jnp
from jax import lax
from jax.experimental import pallas as pl
from jax.experimental.pallas import tpu as pltpu
from jax.experimental.pallas import tpu_sc as plsc

H = 32
HH = 512
N_NODES = 10000
N_EDGES = 160000
N_LAYER = 2

NW = 32
ROUNDS = 4
NPT = 80
N_PAD = ROUNDS * NW * NPT
CE = 2000
NCHUNK = N_EDGES // CE
G = 48
NFP = 128


def _elu(x):
    return jnp.where(x > 0, x, jnp.exp(jnp.minimum(x, 0.0)) - 1.0)


def _ln(x, g, b, eps=1e-5):
    mu = jnp.mean(x, axis=-1, keepdims=True)
    xc = x - mu
    var = jnp.mean(xc * xc, axis=-1, keepdims=True)
    return xc * jax.lax.rsqrt(var + eps) * g + b


def _embed_body(x_ref, w_ref, b_ref, o_ref):
    o_ref[...] = _elu(
        jnp.dot(x_ref[...], w_ref[...], preferred_element_type=jnp.float32)
        + b_ref[...])


def _embed(x, w, b, tile):
    n = x.shape[0]
    return pl.pallas_call(
        _embed_body,
        grid=(n // tile,),
        in_specs=[
            pl.BlockSpec((tile, x.shape[1]), lambda i: (i, 0)),
            pl.BlockSpec(w.shape, lambda i: (0, 0)),
            pl.BlockSpec((1, H), lambda i: (0, 0)),
        ],
        out_specs=pl.BlockSpec((tile, H), lambda i: (i, 0)),
        out_shape=jax.ShapeDtypeStruct((n, H), jnp.float32),
    )(x, w, b[None, :])


def _ln_body(x_ref, g_ref, b_ref, o_ref):
    y = _ln(x_ref[...], g_ref[...], b_ref[...])
    o_ref[...] = jnp.concatenate(
        [y, jnp.zeros((y.shape[0], NFP - H), jnp.float32)], axis=1)


def _ln_call(x, g, b, tile):
    n = x.shape[0]
    return pl.pallas_call(
        _ln_body,
        grid=(n // tile,),
        in_specs=[
            pl.BlockSpec((tile, H), lambda i: (i, 0)),
            pl.BlockSpec((1, H), lambda i: (0, 0)),
            pl.BlockSpec((1, H), lambda i: (0, 0)),
        ],
        out_specs=pl.BlockSpec((tile, NFP), lambda i: (i, 0)),
        out_shape=jax.ShapeDtypeStruct((n, NFP), jnp.float32),
    )(x, g[None, :], b[None, :])


def _wmat_body(he_ref, lng_ref, lnb_ref, wm_ref, bm_ref, *o_refs):
    x = he_ref[...]
    for l in range(N_LAYER):
        ef = _ln(x, lng_ref[l][None, :], lnb_ref[l][None, :])
        for d in range(2):
            k = 2 * l + d
            o_refs[k][...] = (
                jnp.dot(ef, wm_ref[k], preferred_element_type=jnp.float32)
                + bm_ref[k][None, :])


def _wmat(h_e, lng, lnb, wm, bm, tile):
    return pl.pallas_call(
        _wmat_body,
        grid=(N_EDGES // tile,),
        in_specs=[
            pl.BlockSpec((tile, H), lambda i: (i, 0)),
            pl.BlockSpec((N_LAYER, H), lambda i: (0, 0)),
            pl.BlockSpec((N_LAYER, H), lambda i: (0, 0)),
            pl.BlockSpec((4, H, HH), lambda i: (0, 0, 0)),
            pl.BlockSpec((4, HH), lambda i: (0, 0)),
        ],
        out_specs=[pl.BlockSpec((tile, HH), lambda i: (i, 0))] * 4,
        out_shape=[jax.ShapeDtypeStruct((N_EDGES, HH), jnp.float32)] * 4,
    )(h_e, lng, lnb, wm, bm)


def _post_body(ni_ref, no_ref, nf_ref, sl_ref, sr_ref, bc_ref, mg_ref,
               mb_ref, o_ref):
    neg_inf = jnp.float32(-jnp.inf)
    ni = ni_ref[...]
    no = no_ref[...]
    ni = jnp.where(ni == neg_inf, 0.0, ni)
    no = jnp.where(no == neg_inf, 0.0, no)
    m = (jnp.dot(ni, sl_ref[...], preferred_element_type=jnp.float32)
         + jnp.dot(no, sr_ref[...], preferred_element_type=jnp.float32)
         + bc_ref[...])
    m = _ln(m, mg_ref[...], mb_ref[...])
    o_ref[...] = _elu(nf_ref[...][:, :H] + m)


def _post(ni, no, nf, sl, sr, bcat, mg, mb, tile):
    return pl.pallas_call(
        _post_body,
        grid=(N_NODES // tile,),
        in_specs=[
            pl.BlockSpec((tile, HH), lambda i: (i, 0)),
            pl.BlockSpec((tile, HH), lambda i: (i, 0)),
            pl.BlockSpec((tile, NFP), lambda i: (i, 0)),
            pl.BlockSpec((HH, H), lambda i: (0, 0)),
            pl.BlockSpec((HH, H), lambda i: (0, 0)),
            pl.BlockSpec((1, H), lambda i: (0, 0)),
            pl.BlockSpec((1, H), lambda i: (0, 0)),
            pl.BlockSpec((1, H), lambda i: (0, 0)),
        ],
        out_specs=pl.BlockSpec((tile, H), lambda i: (i, 0)),
        out_shape=jax.ShapeDtypeStruct((N_NODES, H), jnp.float32),
    )(ni, no, nf, sl, sr, bcat[None, :], mg[None, :], mb[None, :])


def _pool_body(h_ref, o_ref):
    i = pl.program_id(0)
    t = jnp.max(h_ref[...], axis=0, keepdims=True)

    @pl.when(i == 0)
    def _():
        o_ref[...] = t

    @pl.when(i > 0)
    def _():
        o_ref[...] = jnp.maximum(o_ref[...], t)


def _pool(h_n, tile):
    return pl.pallas_call(
        _pool_body,
        grid=(N_NODES // tile,),
        in_specs=[pl.BlockSpec((tile, H), lambda i: (i, 0))],
        out_specs=pl.BlockSpec((1, H), lambda i: (0, 0)),
        out_shape=jax.ShapeDtypeStruct((1, H), jnp.float32),
    )(h_n)


def _mlp_body(gf_ref, w1_ref, b1_ref, w2_ref, b2_ref, out_ref):
    hmid = _elu(
        jnp.dot(gf_ref[...], w1_ref[...], preferred_element_type=jnp.float32)
        + b1_ref[...])
    out_ref[...] = (
        jnp.dot(hmid, w2_ref[...], preferred_element_type=jnp.float32)
        + b2_ref[...])


def _sc_scatter_max(w_all, agg_idx, oth_idx, nfeat):
    mesh = plsc.VectorSubcoreMesh(core_axis_name="c", subcore_axis_name="s")

    @functools.partial(
        pl.kernel,
        mesh=mesh,
        compiler_params=pltpu.CompilerParams(needs_layout_passes=False),
        out_type=jax.ShapeDtypeStruct((N_PAD, HH), jnp.float32),
        scratch_types=[
            pltpu.VMEM((NPT, HH), jnp.float32),
            pltpu.VMEM((CE,), jnp.int32),
            pltpu.VMEM((CE,), jnp.int32),
            pltpu.VMEM((CE,), jnp.int32),
            pltpu.VMEM((CE,), jnp.int32),
            pltpu.VMEM((CE + G,), jnp.int32),
            pltpu.VMEM((CE + G,), jnp.int32),
            pltpu.VMEM((CE + G,), jnp.int32),
            pltpu.VMEM((CE + G,), jnp.int32),
            pltpu.VMEM((CE + G,), jnp.int32),
            pltpu.VMEM((CE + G,), jnp.int32),
            pltpu.VMEM((G, HH), jnp.float32),
            pltpu.VMEM((G, HH), jnp.float32),
            pltpu.VMEM((G, NFP), jnp.float32),
            pltpu.VMEM((G, NFP), jnp.float32),
            pltpu.SemaphoreType.DMA,
            pltpu.SemaphoreType.DMA,
            pltpu.SemaphoreType.DMA,
            pltpu.SemaphoreType.DMA,
            pltpu.SemaphoreType.DMA,
            pltpu.SemaphoreType.DMA,
        ],
    )
    def k(w_hbm, agg_hbm, oth_hbm, nf_hbm, out_hbm,
          acc, aggc0, aggc1, othc0, othc1,
          eidm0, eidm1, othm0, othm1, aggm0, aggm1,
          wr0, wr1, nfr0, nfr1, si0, si1, gw0, gw1, gn0, gn1):
        aggc = (aggc0, aggc1)
        othc = (othc0, othc1)
        eidm = (eidm0, eidm1)
        othm = (othm0, othm1)
        aggm = (aggm0, aggm1)
        wr = (wr0, wr1)
        nfr = (nfr0, nfr1)
        si = (si0, si1)
        gw = (gw0, gw1)
        gn = (gn0, gn1)

        wid = lax.axis_index("s") * 2 + lax.axis_index("c")
        neg = jnp.full((16,), -jnp.inf, dtype=jnp.float32)
        zero = jnp.zeros((16,), dtype=jnp.int32)
        ioto = lax.iota(jnp.int32, 16)

        def _zi(i, _):
            for b in range(2):
                eidm[b][pl.ds(i * 16, 16)] = zero
                othm[b][pl.ds(i * 16, 16)] = zero
            return 0
        lax.fori_loop(0, (CE + G) // 16, _zi, 0)

        def issue_idx(b, cb):
            pltpu.async_copy(agg_hbm.at[pl.ds(cb, CE)], aggc[b], si[b])
            pltpu.async_copy(oth_hbm.at[pl.ds(cb, CE)], othc[b], si[b])

        def wait_idx(b):
            pltpu.make_async_copy(
                agg_hbm.at[pl.ds(0, CE)], aggc[b], si[b]).wait()
            pltpu.make_async_copy(
                oth_hbm.at[pl.ds(0, CE)], othc[b], si[b]).wait()

        def issue_gather(b, gb):
            pltpu.async_copy(
                w_hbm.at[eidm[b].at[pl.ds(gb, G)]], wr[b], gw[b])
            pltpu.async_copy(
                nf_hbm.at[othm[b].at[pl.ds(gb, G)]], nfr[b], gn[b])

        def wait_gather(b):
            pltpu.make_async_copy(
                w_hbm.at[pl.ds(0, G)], wr[b], gw[b]).wait()
            pltpu.make_async_copy(
                nf_hbm.at[pl.ds(0, G)], nfr[b], gn[b]).wait()

        def scan(b, cbase, lov, hiv):
            def _s(i, cnt):
                v = aggc[b][pl.ds(i * 16, 16)]
                m = (v >= lov) & (v < hiv)
                pref = plsc.cumsum(m.astype(jnp.int32))
                pos = lax.broadcast(cnt, (16,)) + pref - 1
                eid = ioto + lax.broadcast(cbase + i * 16, (16,))
                plsc.store_scatter(eidm[b], [pos], eid, mask=m)
                ov = othc[b][pl.ds(i * 16, 16)]
                plsc.store_scatter(othm[b], [pos], ov, mask=m)
                plsc.store_scatter(aggm[b], [pos], v - lov, mask=m)
                return cnt + pref[15]
            return lax.fori_loop(0, CE // 16, _s, jnp.int32(0))

        def accum(b, gb, rem):
            def _edge(j, _):
                a = aggm[b][pl.ds(gb + j, 16)][0]
                nf0 = nfr[b][j, pl.ds(0, 16)]
                nf1 = nfr[b][j, pl.ds(16, 16)]
                for hh in range(H):
                    nfs = nf0[hh] if hh < 16 else nf1[hh - 16]
                    wv = wr[b][j, pl.ds(hh * 16, 16)]
                    nfv = lax.broadcast(nfs, (16,))
                    cur = acc[a, pl.ds(hh * 16, 16)]
                    acc[a, pl.ds(hh * 16, 16)] = jnp.maximum(cur, wv * nfv)
                return 0
            lax.fori_loop(0, rem, _edge, 0)

        def drain(b, pend):
            wait_gather(b)
            accum(b, 0, jnp.minimum(pend, G))
            ng = (pend + (G - 1)) // G

            def _g(g, _):
                gb = g * G
                issue_gather(b, gb)
                wait_gather(b)
                accum(b, gb, jnp.minimum(pend - gb, G))
                return 0
            lax.fori_loop(1, ng, _g, 0)

        issue_idx(0, 0)

        def _round(r, _):
            base = (r * NW + wid) * NPT
            lov = lax.broadcast(base, (16,))
            hiv = lax.broadcast(base + NPT, (16,))

            def _init(i, _):
                for hh in range(HH // 16):
                    acc[i, pl.ds(hh * 16, 16)] = neg
                return 0
            lax.fori_loop(0, NPT, _init, 0)

            issue_gather(1, 0)

            def _step(b, c, pend):
                wait_idx(b)
                nb = jnp.where(c + 1 < NCHUNK, c + 1, 0)
                issue_idx(1 - b, nb * CE)
                cnt = scan(b, c * CE, lov, hiv)
                issue_gather(b, 0)
                drain(1 - b, pend)
                return cnt

            def _cc(cc, pend):
                pend = _step(0, 2 * cc, pend)
                pend = _step(1, 2 * cc + 1, pend)
                return pend

            pend = lax.fori_loop(0, NCHUNK // 2, _cc, jnp.int32(0))
            drain(1, pend)
            pltpu.sync_copy(acc, out_hbm.at[pl.ds(base, NPT)])
            return 0

        lax.fori_loop(0, ROUNDS, _round, 0)
        wait_idx(0)

    return k(w_all, agg_idx, oth_idx, nfeat)


def kernel(node_inp, edge_inp, params, edge_index):
    p = params
    src = edge_index[0]
    dst = edge_index[1]

    def pad13(x):
        return jnp.pad(x, ((0, 0), (0, 3)))

    h_n = _embed(pad13(node_inp), jnp.pad(p['Wn'], ((0, 3), (0, 0))),
                 p['bn'], 1000)
    h_e = _embed(pad13(edge_inp), jnp.pad(p['We'], ((0, 3), (0, 0))),
                 p['be'], 1000)

    lng = jnp.stack([p[f'ln_g{l}'] for l in range(N_LAYER)])
    lnb = jnp.stack([p[f'ln_b{l}'] for l in range(N_LAYER)])
    wm = jnp.stack([p['Wmi0'], p['Wmo0'], p['Wmi1'], p['Wmo1']])
    bm = jnp.stack([p['bmi0'], p['bmo0'], p['bmi1'], p['bmo1']])
    wi0, wo0, wi1, wo1 = _wmat(h_e, lng, lnb, wm, bm, 1000)
    w_by_layer = ((wi0, wo0), (wi1, wo1))

    eye = jnp.eye(H // 2, dtype=jnp.float32)
    si = jnp.tile(eye, (H, 1))
    zz = jnp.zeros((HH, H // 2), jnp.float32)
    sl = jnp.concatenate([si, zz], axis=1)
    sr = jnp.concatenate([zz, si], axis=1)

    for l in range(N_LAYER):
        nfeat = _ln_call(h_n, p[f'ln_g{l}'], p[f'ln_b{l}'], 1000)
        wi, wo = w_by_layer[l]
        acc_i = _sc_scatter_max(wi, dst, src, nfeat)
        acc_o = _sc_scatter_max(wo, src, dst, nfeat)
        bcat = jnp.concatenate([p[f'bias_i{l}'], p[f'bias_o{l}']])
        h_n = _post(acc_i[:N_NODES], acc_o[:N_NODES], nfeat, sl, sr, bcat,
                    p[f'mn_g{l}'], p[f'mn_b{l}'], 1000)

    gf = _pool(h_n, 1000)
    pred = pl.pallas_call(
        _mlp_body,
        out_shape=jax.ShapeDtypeStruct((1, 1), jnp.float32),
    )(gf, p['W1'], p['b1'][None, :], p['W2'], p['b2'][None, :])
    return pred


# speedup vs baseline: 5.8081x; 5.8081x over previous
import functools

import jax
import jax.numpy as jnp
from jax import lax
from jax.experimental import pallas as pl
from jax.experimental.pallas import tpu as pltpu
from jax.experimental.pallas import tpu_sc as plsc

H = 32
HH = 512
N_NODES = 10000
N_EDGES = 160000
N_LAYER = 2

NW = 32
ROUNDS = 2
NPT = 160
N_PAD = ROUNDS * NW * NPT
CE = 2000
NCHUNK = N_EDGES // CE
G = 48
NFP = 128


def _elu(x):
    return jnp.where(x > 0, x, jnp.exp(jnp.minimum(x, 0.0)) - 1.0)


def _ln(x, g, b, eps=1e-5):
    mu = jnp.mean(x, axis=-1, keepdims=True)
    xc = x - mu
    var = jnp.mean(xc * xc, axis=-1, keepdims=True)
    return xc * jax.lax.rsqrt(var + eps) * g + b


def _embed_body(x_ref, w_ref, b_ref, o_ref):
    o_ref[...] = _elu(
        jnp.dot(x_ref[...], w_ref[...], preferred_element_type=jnp.float32)
        + b_ref[...])


def _embed(x, w, b, tile):
    n = x.shape[0]
    return pl.pallas_call(
        _embed_body,
        grid=(n // tile,),
        in_specs=[
            pl.BlockSpec((tile, x.shape[1]), lambda i: (i, 0)),
            pl.BlockSpec(w.shape, lambda i: (0, 0)),
            pl.BlockSpec((1, H), lambda i: (0, 0)),
        ],
        out_specs=pl.BlockSpec((tile, H), lambda i: (i, 0)),
        out_shape=jax.ShapeDtypeStruct((n, H), jnp.float32),
    )(x, w, b[None, :])


def _ln_body(x_ref, g_ref, b_ref, o_ref):
    y = _ln(x_ref[...], g_ref[...], b_ref[...])
    o_ref[...] = jnp.concatenate(
        [y, jnp.zeros((y.shape[0], NFP - H), jnp.float32)], axis=1)


def _ln_call(x, g, b, tile):
    n = x.shape[0]
    return pl.pallas_call(
        _ln_body,
        grid=(n // tile,),
        in_specs=[
            pl.BlockSpec((tile, H), lambda i: (i, 0)),
            pl.BlockSpec((1, H), lambda i: (0, 0)),
            pl.BlockSpec((1, H), lambda i: (0, 0)),
        ],
        out_specs=pl.BlockSpec((tile, NFP), lambda i: (i, 0)),
        out_shape=jax.ShapeDtypeStruct((n, NFP), jnp.float32),
    )(x, g[None, :], b[None, :])


def _wmat_body(he_ref, lng_ref, lnb_ref, wm_ref, bm_ref, *o_refs):
    x = he_ref[...]
    for l in range(N_LAYER):
        ef = _ln(x, lng_ref[l][None, :], lnb_ref[l][None, :])
        for d in range(2):
            k = 2 * l + d
            o_refs[k][...] = (
                jnp.dot(ef, wm_ref[k], preferred_element_type=jnp.float32)
                + bm_ref[k][None, :])


def _wmat(h_e, lng, lnb, wm, bm, tile):
    return pl.pallas_call(
        _wmat_body,
        grid=(N_EDGES // tile,),
        in_specs=[
            pl.BlockSpec((tile, H), lambda i: (i, 0)),
            pl.BlockSpec((N_LAYER, H), lambda i: (0, 0)),
            pl.BlockSpec((N_LAYER, H), lambda i: (0, 0)),
            pl.BlockSpec((4, H, HH), lambda i: (0, 0, 0)),
            pl.BlockSpec((4, HH), lambda i: (0, 0)),
        ],
        out_specs=[pl.BlockSpec((tile, HH), lambda i: (i, 0))] * 4,
        out_shape=[jax.ShapeDtypeStruct((N_EDGES, HH), jnp.float32)] * 4,
    )(h_e, lng, lnb, wm, bm)


def _post_body(ni_ref, no_ref, nf_ref, sl_ref, sr_ref, bc_ref, mg_ref,
               mb_ref, o_ref):
    neg_inf = jnp.float32(-jnp.inf)
    ni = ni_ref[...]
    no = no_ref[...]
    ni = jnp.where(ni == neg_inf, 0.0, ni)
    no = jnp.where(no == neg_inf, 0.0, no)
    m = (jnp.dot(ni, sl_ref[...], preferred_element_type=jnp.float32)
         + jnp.dot(no, sr_ref[...], preferred_element_type=jnp.float32)
         + bc_ref[...])
    m = _ln(m, mg_ref[...], mb_ref[...])
    o_ref[...] = _elu(nf_ref[...][:, :H] + m)


def _post(ni, no, nf, sl, sr, bcat, mg, mb, tile):
    return pl.pallas_call(
        _post_body,
        grid=(N_NODES // tile,),
        in_specs=[
            pl.BlockSpec((tile, HH), lambda i: (i, 0)),
            pl.BlockSpec((tile, HH), lambda i: (i, 0)),
            pl.BlockSpec((tile, NFP), lambda i: (i, 0)),
            pl.BlockSpec((HH, H), lambda i: (0, 0)),
            pl.BlockSpec((HH, H), lambda i: (0, 0)),
            pl.BlockSpec((1, H), lambda i: (0, 0)),
            pl.BlockSpec((1, H), lambda i: (0, 0)),
            pl.BlockSpec((1, H), lambda i: (0, 0)),
        ],
        out_specs=pl.BlockSpec((tile, H), lambda i: (i, 0)),
        out_shape=jax.ShapeDtypeStruct((N_NODES, H), jnp.float32),
    )(ni, no, nf, sl, sr, bcat[None, :], mg[None, :], mb[None, :])


def _pool_body(h_ref, o_ref):
    i = pl.program_id(0)
    t = jnp.max(h_ref[...], axis=0, keepdims=True)

    @pl.when(i == 0)
    def _():
        o_ref[...] = t

    @pl.when(i > 0)
    def _():
        o_ref[...] = jnp.maximum(o_ref[...], t)


def _pool(h_n, tile):
    return pl.pallas_call(
        _pool_body,
        grid=(N_NODES // tile,),
        in_specs=[pl.BlockSpec((tile, H), lambda i: (i, 0))],
        out_specs=pl.BlockSpec((1, H), lambda i: (0, 0)),
        out_shape=jax.ShapeDtypeStruct((1, H), jnp.float32),
    )(h_n)


def _mlp_body(gf_ref, w1_ref, b1_ref, w2_ref, b2_ref, out_ref):
    hmid = _elu(
        jnp.dot(gf_ref[...], w1_ref[...], preferred_element_type=jnp.float32)
        + b1_ref[...])
    out_ref[...] = (
        jnp.dot(hmid, w2_ref[...], preferred_element_type=jnp.float32)
        + b2_ref[...])


def _sc_scatter_max(w_all, agg_idx, oth_idx, nfeat):
    mesh = plsc.VectorSubcoreMesh(core_axis_name="c", subcore_axis_name="s")

    @functools.partial(
        pl.kernel,
        mesh=mesh,
        compiler_params=pltpu.CompilerParams(needs_layout_passes=False),
        out_type=jax.ShapeDtypeStruct((N_PAD, HH), jnp.float32),
        scratch_types=[
            pltpu.VMEM((NPT, HH), jnp.float32),
            pltpu.VMEM((CE,), jnp.int32),
            pltpu.VMEM((CE,), jnp.int32),
            pltpu.VMEM((CE + G,), jnp.int32),
            pltpu.VMEM((CE + G,), jnp.int32),
            pltpu.VMEM((CE + G,), jnp.int32),
            pltpu.VMEM((G, HH), jnp.float32),
            pltpu.VMEM((G, NFP), jnp.float32),
            pltpu.SemaphoreType.DMA,
            pltpu.SemaphoreType.DMA,
            pltpu.SemaphoreType.DMA,
        ],
    )
    def k(w_hbm, agg_hbm, oth_hbm, nf_hbm, out_hbm,
          acc, aggc, othc, eidm, othm, aggm, wr, nfr, si, gw, gn):
        wid = lax.axis_index("s") * 2 + lax.axis_index("c")
        neg = jnp.full((16,), -jnp.inf, dtype=jnp.float32)
        zero = jnp.zeros((16,), dtype=jnp.int32)
        ioto = lax.iota(jnp.int32, 16)

        def _zi(i, _):
            eidm[pl.ds(i * 16, 16)] = zero
            othm[pl.ds(i * 16, 16)] = zero
            return 0
        lax.fori_loop(0, (CE + G) // 16, _zi, 0)

        def wait_idx():
            pltpu.make_async_copy(
                agg_hbm.at[pl.ds(0, CE)], aggc, si).wait()
            pltpu.make_async_copy(
                oth_hbm.at[pl.ds(0, CE)], othc, si).wait()

        def issue_gather(gb):
            pltpu.async_copy(w_hbm.at[eidm.at[pl.ds(gb, G)]], wr, gw)
            pltpu.async_copy(nf_hbm.at[othm.at[pl.ds(gb, G)]], nfr, gn)

        def wait_gather():
            pltpu.make_async_copy(w_hbm.at[pl.ds(0, G)], wr, gw).wait()
            pltpu.make_async_copy(nf_hbm.at[pl.ds(0, G)], nfr, gn).wait()

        def scan(cbase, lov, hiv):
            def _s(i, cnt):
                v = aggc[pl.ds(i * 16, 16)]
                m = (v >= lov) & (v < hiv)
                pref = plsc.cumsum(m.astype(jnp.int32))
                pos = lax.broadcast(cnt, (16,)) + pref - 1
                eid = ioto + lax.broadcast(cbase + i * 16, (16,))
                plsc.store_scatter(eidm, [pos], eid, mask=m)
                ov = othc[pl.ds(i * 16, 16)]
                plsc.store_scatter(othm, [pos], ov, mask=m)
                plsc.store_scatter(aggm, [pos], v - lov, mask=m)
                return cnt + pref[15]
            return lax.fori_loop(0, CE // 16, _s, jnp.int32(0))

        def accum(gb, rem):
            def _edge(j, _):
                a = aggm[pl.ds(gb + j, 16)][0]
                nf0 = nfr[j, pl.ds(0, 16)]
                nf1 = nfr[j, pl.ds(16, 16)]
                for hh in range(H):
                    nfs = nf0[hh] if hh < 16 else nf1[hh - 16]
                    wv = wr[j, pl.ds(hh * 16, 16)]
                    nfv = lax.broadcast(nfs, (16,))
                    cur = acc[a, pl.ds(hh * 16, 16)]
                    acc[a, pl.ds(hh * 16, 16)] = jnp.maximum(cur, wv * nfv)
                return 0
            lax.fori_loop(0, rem, _edge, 0)

        def _round(r, _):
            base = (r * NW + wid) * NPT
            lov = lax.broadcast(base, (16,))
            hiv = lax.broadcast(base + NPT, (16,))

            def _init(i, _):
                for hh in range(HH // 16):
                    acc[i, pl.ds(hh * 16, 16)] = neg
                return 0
            lax.fori_loop(0, NPT, _init, 0)

            def _chunk(c, _):
                pltpu.async_copy(agg_hbm.at[pl.ds(c * CE, CE)], aggc, si)
                pltpu.async_copy(oth_hbm.at[pl.ds(c * CE, CE)], othc, si)
                wait_idx()
                cnt = scan(c * CE, lov, hiv)
                ng = (cnt + (G - 1)) // G

                def _g(g, _):
                    gb = g * G
                    issue_gather(gb)
                    wait_gather()
                    accum(gb, jnp.minimum(cnt - gb, G))
                    return 0
                lax.fori_loop(0, ng, _g, 0)
                return 0

            lax.fori_loop(0, NCHUNK, _chunk, 0)
            pltpu.sync_copy(acc, out_hbm.at[pl.ds(base, NPT)])
            return 0

        lax.fori_loop(0, ROUNDS, _round, 0)

    return k(w_all, agg_idx, oth_idx, nfeat)


def kernel(node_inp, edge_inp, params, edge_index):
    p = params
    src = edge_index[0]
    dst = edge_index[1]

    def pad13(x):
        return jnp.pad(x, ((0, 0), (0, 3)))

    h_n = _embed(pad13(node_inp), jnp.pad(p['Wn'], ((0, 3), (0, 0))),
                 p['bn'], 1000)
    h_e = _embed(pad13(edge_inp), jnp.pad(p['We'], ((0, 3), (0, 0))),
                 p['be'], 1000)

    lng = jnp.stack([p[f'ln_g{l}'] for l in range(N_LAYER)])
    lnb = jnp.stack([p[f'ln_b{l}'] for l in range(N_LAYER)])
    wm = jnp.stack([p['Wmi0'], p['Wmo0'], p['Wmi1'], p['Wmo1']])
    bm = jnp.stack([p['bmi0'], p['bmo0'], p['bmi1'], p['bmo1']])
    wi0, wo0, wi1, wo1 = _wmat(h_e, lng, lnb, wm, bm, 1000)
    w_by_layer = ((wi0, wo0), (wi1, wo1))

    eye = jnp.eye(H // 2, dtype=jnp.float32)
    si = jnp.tile(eye, (H, 1))
    zz = jnp.zeros((HH, H // 2), jnp.float32)
    sl = jnp.concatenate([si, zz], axis=1)
    sr = jnp.concatenate([zz, si], axis=1)

    for l in range(N_LAYER):
        nfeat = _ln_call(h_n, p[f'ln_g{l}'], p[f'ln_b{l}'], 1000)
        wi, wo = w_by_layer[l]
        acc_i = _sc_scatter_max(wi, dst, src, nfeat)
        acc_o = _sc_scatter_max(wo, src, dst, nfeat)
        bcat = jnp.concatenate([p[f'bias_i{l}'], p[f'bias_o{l}']])
        h_n = _post(acc_i[:N_NODES], acc_o[:N_NODES], nfeat, sl, sr, bcat,
                    p[f'mn_g{l}'], p[f'mn_b{l}'], 1000)

    gf = _pool(h_n, 1000)
    pred = pl.pallas_call(
        _mlp_body,
        out_shape=jax.ShapeDtypeStruct((1, 1), jnp.float32),
    )(gf, p['W1'], p['b1'][None, :], p['W2'], p['b2'][None, :])
    return pred
